# baseline (speedup 1.0000x reference)
import functools

import jax
import jax.numpy as jnp
from jax.experimental import pallas as pl
from jax.experimental.pallas import tpu as pltpu

_U = 16


def _body(idx_ref, hidden_ref, weight_ref, w2v_ref, out_ref,
          ksraw, kstile, sems, semr, *, T, K, E, G2):
    c = pl.program_id(0)
    j = pl.program_id(1)
    b = c * G2 + j
    TK = T * K

    def issue(b_, slot):
        base = b_ * TK
        def go(jo, _):
            j0 = jo * _U
            for u in range(_U):
                r = idx_ref[base + j0 + u]
                pltpu.make_async_copy(w2v_ref.at[r], ksraw.at[slot, j0 + u],
                                      sems.at[slot]).start()
            return 0
        jax.lax.fori_loop(0, TK // _U, go, 0, unroll=False)

    slot = jax.lax.rem(j, 2)

    @pl.when(j == 0)
    def _():
        issue(b, slot)

    @pl.when(j + 1 < G2)
    def _():
        issue(b + 1, 1 - slot)

    q = jnp.dot(hidden_ref[...], weight_ref[...],
                preferred_element_type=jnp.float32)

    pltpu.make_async_copy(w2v_ref.at[pl.ds(0, TK)], ksraw.at[slot],
                          sems.at[slot]).wait()

    cp = pltpu.make_async_copy(ksraw.at[slot, pl.ds(0, TK), 0], kstile, semr)
    cp.start()
    cp.wait()

    ks3 = kstile[...].reshape(T, K, E)
    ksf = ks3.astype(jnp.bfloat16).astype(jnp.float32)
    qf = q.astype(jnp.bfloat16).astype(jnp.float32)
    scores = jnp.sum(ksf * qf[:, None, :], axis=-1)
    m = jnp.max(scores, axis=-1, keepdims=True)
    p = jnp.exp(scores - m)
    attn = p / jnp.sum(p, axis=-1, keepdims=True)
    attn = attn.astype(jnp.bfloat16).astype(jnp.float32)
    out_ref[...] = jnp.sum(ksf * attn[:, :, None], axis=1)


def kernel(hidden_feature, similar_words_sent, word2vec, weight):
    B, L, E = hidden_feature.shape
    K = similar_words_sent.shape[-1]
    V = word2vec.shape[0]
    N = B * L
    T = 256 if N % 512 == 0 else N // 2
    G = N // T
    G2 = G // 2

    idx = similar_words_sent.reshape(N * K).astype(jnp.int32)
    hidden = hidden_feature.reshape(N, E)
    w2v3 = word2vec.reshape(V, 1, E)

    grid_spec = pltpu.PrefetchScalarGridSpec(
        num_scalar_prefetch=1,
        grid=(2, G2),
        in_specs=[
            pl.BlockSpec((T, E), lambda c, j, idx_ref: (c * G2 + j, 0)),
            pl.BlockSpec((E, E), lambda c, j, idx_ref: (0, 0)),
            pl.BlockSpec(memory_space=pl.ANY),
        ],
        out_specs=pl.BlockSpec((T, E), lambda c, j, idx_ref: (c * G2 + j, 0)),
        scratch_shapes=[
            pltpu.VMEM((2, T * K, 1, E), jnp.float32),
            pltpu.VMEM((T * K, E), jnp.float32),
            pltpu.SemaphoreType.DMA((2,)),
            pltpu.SemaphoreType.DMA,
        ],
    )
    out = pl.pallas_call(
        functools.partial(_body, T=T, K=K, E=E, G2=G2),
        grid_spec=grid_spec,
        out_shape=jax.ShapeDtypeStruct((N, E), jnp.float32),
        compiler_params=pltpu.CompilerParams(
            dimension_semantics=("arbitrary", "arbitrary"),
        ),
    )(idx, hidden, weight, w2v3)
    return out.reshape(B, L, E)


# speedup vs baseline: 1.0294x; 1.0294x over previous
import functools

import jax
import jax.numpy as jnp
from jax.experimental import pallas as pl
from jax.experimental.pallas import tpu as pltpu

_U = 16


def _body(idx_ref, hidden_ref, weight_ref, w2v_ref, out_ref,
          ksraw, kstile, sems, semr, *, T, K, E, G2):
    c = pl.program_id(0)
    j = pl.program_id(1)
    b = c * G2 + j
    TK = T * K

    def issue(b_, slot):
        base = b_ * TK
        def go(jo, _):
            j0 = jo * _U
            for u in range(_U):
                r = idx_ref[base + j0 + u]
                pltpu.make_async_copy(w2v_ref.at[r], ksraw.at[slot, j0 + u],
                                      sems.at[slot]).start()
            return 0
        jax.lax.fori_loop(0, TK // _U, go, 0, unroll=False)

    slot = jax.lax.rem(j, 2)

    @pl.when(j == 0)
    def _():
        issue(b, slot)

    @pl.when(j + 1 < G2)
    def _():
        issue(b + 1, 1 - slot)

    q = jnp.dot(hidden_ref[...], weight_ref[...],
                preferred_element_type=jnp.float32)

    pltpu.make_async_copy(w2v_ref.at[pl.ds(0, TK)], ksraw.at[slot],
                          sems.at[slot]).wait()

    cp = pltpu.make_async_copy(ksraw.at[slot, pl.ds(0, TK), 0], kstile, semr)
    cp.start()
    cp.wait()

    ks3 = kstile[...].reshape(T, K, E)
    ksf = ks3.astype(jnp.bfloat16).astype(jnp.float32)
    qf = q.astype(jnp.bfloat16).astype(jnp.float32)
    scores = jnp.sum(ksf * qf[:, None, :], axis=-1)
    m = jnp.max(scores, axis=-1, keepdims=True)
    p = jnp.exp(scores - m)
    attn = p / jnp.sum(p, axis=-1, keepdims=True)
    attn = attn.astype(jnp.bfloat16).astype(jnp.float32)
    out_ref[...] = jnp.sum(ksf * attn[:, :, None], axis=1)


def kernel(hidden_feature, similar_words_sent, word2vec, weight):
    B, L, E = hidden_feature.shape
    K = similar_words_sent.shape[-1]
    V = word2vec.shape[0]
    N = B * L
    T = 128 if N % 256 == 0 else N // 2
    G = N // T
    G2 = G // 2

    idx = similar_words_sent.reshape(N * K).astype(jnp.int32)
    hidden = hidden_feature.reshape(N, E)
    w2v3 = word2vec.reshape(V, 1, E)

    grid_spec = pltpu.PrefetchScalarGridSpec(
        num_scalar_prefetch=1,
        grid=(2, G2),
        in_specs=[
            pl.BlockSpec((T, E), lambda c, j, idx_ref: (c * G2 + j, 0)),
            pl.BlockSpec((E, E), lambda c, j, idx_ref: (0, 0)),
            pl.BlockSpec(memory_space=pl.ANY),
        ],
        out_specs=pl.BlockSpec((T, E), lambda c, j, idx_ref: (c * G2 + j, 0)),
        scratch_shapes=[
            pltpu.VMEM((2, T * K, 1, E), jnp.float32),
            pltpu.VMEM((T * K, E), jnp.float32),
            pltpu.SemaphoreType.DMA((2,)),
            pltpu.SemaphoreType.DMA,
        ],
    )
    out = pl.pallas_call(
        functools.partial(_body, T=T, K=K, E=E, G2=G2),
        grid_spec=grid_spec,
        out_shape=jax.ShapeDtypeStruct((N, E), jnp.float32),
        compiler_params=pltpu.CompilerParams(
            dimension_semantics=("parallel", "arbitrary"),
        ),
    )(idx, hidden, weight, w2v3)
    return out.reshape(B, L, E)
